# baseline (speedup 1.0000x reference)
import functools

import jax
import jax.numpy as jnp
from jax import lax
from jax.experimental import pallas as pl
from jax.experimental.pallas import tpu as pltpu
from jax.experimental.pallas import tpu_sc as plsc

_N_E = 1024
_E_DIM = 256
_BETA = 0.25
_TOK_TILE = 1024
_BATCH = 16
_N_TOK = _BATCH * _TOK_TILE

_SC_CORES = 2
_SC_SUBCORES = 16
_NW = _SC_CORES * _SC_SUBCORES
_CH_PER_W = _E_DIM // _NW
_LANES = 16


def _dist_argmin_kernel(z_ref, w_ref, idx_ref, loss_ref, w2_ref):
    zt = z_ref[0]
    zf = zt.T
    wv = w_ref[...]
    i = pl.program_id(0)

    @pl.when(i == 0)
    def _once():
        w2_ref[...] = jnp.sum(wv * wv, axis=1, keepdims=True)
        loss_ref[...] = jnp.zeros_like(loss_ref)

    z2r = jnp.sum(zt * zt, axis=0, keepdims=True)
    mm = jnp.dot(wv, zt, preferred_element_type=jnp.float32)
    d = (z2r + w2_ref[...]) - 2.0 * mm

    dmin = jnp.min(d, axis=0, keepdims=True)
    rowf = jax.lax.broadcasted_iota(jnp.int32, d.shape, 0).astype(jnp.float32)
    idxf = jnp.min(jnp.where(d == dmin, rowf, jnp.float32(_N_E)),
                   axis=0, keepdims=True)
    idx_ref[0] = idxf.astype(jnp.int32)

    loss_ref[0, :] += jnp.sum(dmin[0].reshape(8, 128), axis=0)


@functools.partial(
    pl.kernel,
    mesh=plsc.VectorSubcoreMesh(core_axis_name="c", subcore_axis_name="s",
                                num_cores=_SC_CORES, num_subcores=_SC_SUBCORES),
    out_type=jax.ShapeDtypeStruct((_BATCH, _E_DIM, _TOK_TILE), jnp.float32),
    scratch_types=[
        pltpu.VMEM((_CH_PER_W * _N_E,), jnp.float32),
        pltpu.VMEM((_N_TOK,), jnp.int32),
        pltpu.VMEM((_CH_PER_W, _TOK_TILE), jnp.float32),
    ],
    compiler_params=pltpu.CompilerParams(needs_layout_passes=False),
)
def _sc_gather(wt_hbm, idx_hbm, out_hbm, wt_v, idx_v, buf_v):
    wid = lax.axis_index("s") * _SC_CORES + lax.axis_index("c")
    c0 = wid * _CH_PER_W
    pltpu.sync_copy(wt_hbm.at[pl.ds(c0 * _N_E, _CH_PER_W * _N_E)], wt_v)
    pltpu.sync_copy(idx_hbm, idx_v)
    for b in range(_BATCH):
        def body(k, _, b=b):
            idxvec = idx_v[pl.ds(b * _TOK_TILE + k * _LANES, _LANES)]
            for c in range(_CH_PER_W):
                buf_v[c, pl.ds(k * _LANES, _LANES)] = plsc.load_gather(
                    wt_v, [idxvec + jnp.int32(c * _N_E)])
            return 0
        lax.fori_loop(0, _TOK_TILE // _LANES, body, 0)
        pltpu.sync_copy(buf_v, out_hbm.at[b, pl.ds(c0, _CH_PER_W)])


def kernel(z, W):
    b, c, h, w = z.shape
    hw = h * w
    z3 = z.reshape(b, c, hw)

    idx3, lossvec = pl.pallas_call(
        _dist_argmin_kernel,
        grid=(b,),
        in_specs=[
            pl.BlockSpec((1, _E_DIM, _TOK_TILE), lambda i: (i, 0, 0)),
            pl.BlockSpec((_N_E, _E_DIM), lambda i: (0, 0)),
        ],
        out_specs=[
            pl.BlockSpec((1, 1, _TOK_TILE), lambda i: (i, 0, 0)),
            pl.BlockSpec((1, 128), lambda i: (0, 0)),
        ],
        out_shape=[
            jax.ShapeDtypeStruct((b, 1, _TOK_TILE), jnp.int32),
            jax.ShapeDtypeStruct((1, 128), jnp.float32),
        ],
        scratch_shapes=[pltpu.VMEM((_N_E, 1), jnp.float32)],
        compiler_params=pltpu.CompilerParams(
            dimension_semantics=("arbitrary",)),
    )(z3, W)

    wt = W.T

    quant3 = _sc_gather(wt.reshape(_E_DIM * _N_E), idx3.reshape(_N_TOK))
    quantize = quant3.reshape(b, c, h, w)
    index = idx3.reshape(b, h, w)
    m = jnp.sum(lossvec) / (b * hw * c)
    loss = m + _BETA * m
    return quantize, loss, index
